# baseline (speedup 1.0000x reference)
import functools

import numpy as np
import jax
import jax.numpy as jnp
from jax import lax
from jax.experimental import pallas as pl
from jax.experimental.pallas import tpu as pltpu
from jax.experimental.pallas import tpu_sc as plsc

_MAX_LEN = 2048

_NUM_CORES = 2
_NUM_SUBCORES = 16
_NUM_WORKERS = _NUM_CORES * _NUM_SUBCORES
_LANES = 16


def _positional_encoding(max_len, d_model):
    pos = np.arange(max_len, dtype=np.float32)[:, None]
    i2 = np.arange(0, d_model, 2, dtype=np.float32)
    div = np.power(10000.0, i2 / d_model)
    pe = np.zeros((max_len, d_model), dtype=np.float32)
    pe[:, 0::2] = np.sin(pos / div)
    pe[:, 1::2] = np.cos(pos / div)
    return jnp.asarray(pe)


@functools.cache
def _build_kernel(N, S, D, C):
    n_per_w = N // _NUM_WORKERS
    n_chunks = n_per_w // C
    mesh = plsc.VectorSubcoreMesh(core_axis_name="c", subcore_axis_name="s")

    @functools.partial(
        pl.kernel,
        out_type=jax.ShapeDtypeStruct((N, D), jnp.float32),
        mesh=mesh,
        scratch_types=[
            pltpu.VMEM((C,), jnp.int32),
            pltpu.VMEM((C, D), jnp.float32),
            pltpu.VMEM((C, D), jnp.float32),
            pltpu.SemaphoreType.DMA,
        ],
    )
    def emb_kernel(x_hbm, table_hbm, pe_hbm, out_hbm, idx_v, rows_v, acc_v, sem):
        wid = lax.axis_index("s") * _NUM_CORES + lax.axis_index("c")
        base = wid * n_per_w
        s0 = base % S

        def chunk_body(c, _):
            off = base + c * C
            pltpu.sync_copy(x_hbm.at[pl.ds(off, C)], idx_v)
            pltpu.sync_copy(pe_hbm.at[pl.ds(s0 + c * C, C)], acc_v)
            pltpu.async_copy(table_hbm.at[idx_v], rows_v, sem).wait()

            def row_body(r, _):
                for j in range(D // _LANES):
                    v = rows_v[r, pl.ds(j * _LANES, _LANES)]
                    plsc.addupdate(acc_v.at[r, pl.ds(j * _LANES, _LANES)], v)
                return ()

            lax.fori_loop(0, C, row_body, (), unroll=False)
            pltpu.sync_copy(acc_v, out_hbm.at[pl.ds(off, C)])
            return ()

        lax.fori_loop(0, n_chunks, chunk_body, (), unroll=False)

    return emb_kernel


def kernel(x, table):
    B, S = x.shape
    _, D = table.shape
    N = B * S
    pe = _positional_encoding(_MAX_LEN, D)[:S]
    x_flat = x.reshape(N).astype(jnp.int32)
    out = _build_kernel(N, S, D, 64)(x_flat, table, pe)
    return out.reshape(B, S, D)
